# Initial kernel scaffold; baseline (speedup 1.0000x reference)
import jax
import jax.numpy as jnp
from jax.experimental import pallas as pl


def kernel(x, conv1_w, conv1_b, conv2_w, conv2_b, rgb0_w, rgb0_b, rgb1_w, rgb1_b, alpha):
    raise NotImplementedError("write your pallas kernel here")


# speedup vs baseline: 1.7305x; 1.7305x over previous
import functools

import jax
import jax.numpy as jnp
from jax.experimental import pallas as pl
from jax.experimental.pallas import tpu as pltpu

_NEG_SLOPE = 0.2
_EPS = 1e-8


def _pixnorm_lrelu(h):
    norm = jnp.sqrt(jnp.sum(h * h, axis=1, keepdims=True))
    h = h * pl.reciprocal(norm + _EPS, approx=True)
    return jnp.maximum(h, _NEG_SLOPE * h)


def _fused_kernel(x_ref, w1e_ref, b1_ref, w2_ref, b2_ref, wr0_ref, wr1_ref,
                  br_ref, o_ref, *, H, W):
    S = H * W
    x = x_ref[...]
    bf = jnp.bfloat16

    j = jax.lax.broadcasted_iota(jnp.int32, (S, 1), 0) % W
    m_hi = (j < (W - 1)).astype(bf)
    m_lo = (j > 0).astype(bf)

    def colshift(v, sx):
        if sx == 0:
            return v
        z = jnp.zeros((1, v.shape[1]), v.dtype)
        if sx == 1:
            return jnp.concatenate([v[1:], z], axis=0) * m_hi
        return jnp.concatenate([z, v[:-1]], axis=0) * m_lo

    def rowshift(v, sy):
        if sy == 0:
            return v
        z = jnp.zeros((W, v.shape[1]), v.dtype)
        if sy == 1:
            return jnp.concatenate([v[W:], z], axis=0)
        return jnp.concatenate([z, v[:S - W]], axis=0)

    cx = {sx: colshift(x, sx) for sx in (-1, 0, 1)}
    h1 = {}
    for p in (0, 1):
        for q in (0, 1):
            acc = None
            for a in (0, 1):
                for b in (0, 1):
                    sy, sx = a - 1 + p, b - 1 + q
                    idx = (p * 2 + q) * 4 + a * 2 + b
                    t = jnp.dot(rowshift(cx[sx], sy), w1e_ref[idx],
                                preferred_element_type=jnp.float32)
                    acc = t if acc is None else acc + t
            h1[(p, q)] = _pixnorm_lrelu(acc + b1_ref[...]).astype(bf)

    ch = {}
    h2 = {}
    for p in (0, 1):
        for q in (0, 1):
            acc = None
            for dy in (-1, 0, 1):
                for dx in (-1, 0, 1):
                    pp, qq = (p + dy) % 2, (q + dx) % 2
                    sy, sx = (p + dy) // 2, (q + dx) // 2
                    key = (pp, qq, sx)
                    if key not in ch:
                        ch[key] = colshift(h1[(pp, qq)], sx)
                    t = jnp.dot(rowshift(ch[key], sy),
                                w2_ref[(dy + 1) * 3 + (dx + 1)],
                                preferred_element_type=jnp.float32)
                    acc = t if acc is None else acc + t
            h2[(p, q)] = _pixnorm_lrelu(acc + b2_ref[...]).astype(bf)

    base = jnp.dot(x, wr0_ref[...], preferred_element_type=jnp.float32) \
        + br_ref[...]
    for ph, (p, q) in enumerate(((0, 0), (0, 1), (1, 0), (1, 1))):
        o_ref[ph * S:(ph + 1) * S, :] = base + jnp.dot(
            h2[(p, q)], wr1_ref[...], preferred_element_type=jnp.float32)


def _const_spec(a):
    return pl.BlockSpec(a.shape, lambda n: (0,) * a.ndim)


def kernel(x, conv1_w, conv1_b, conv2_w, conv2_b,
           rgb0_w, rgb0_b, rgb1_w, rgb1_b, alpha):
    N, C, H, W = x.shape
    Co = conv2_w.shape[3]
    S = H * W
    bf = jnp.bfloat16

    xt = x.reshape(N, C, S).transpose(0, 2, 1).astype(bf)

    w1 = conv1_w.astype(jnp.float32)
    wy = (jnp.stack([w1[0], w1[1] + w1[2]]),
          jnp.stack([w1[0] + w1[1], w1[2]]))
    parts = []
    for p in (0, 1):
        for q in (0, 1):
            wa = wy[p]
            if q == 0:
                e = jnp.stack([wa[:, 0], wa[:, 1] + wa[:, 2]], axis=1)
            else:
                e = jnp.stack([wa[:, 0] + wa[:, 1], wa[:, 2]], axis=1)
            parts.append(e.reshape(4, C, C))
    w1e = jnp.concatenate(parts).astype(bf)

    w2 = conv2_w.reshape(9, C, Co).astype(bf)
    b1 = conv1_b.reshape(1, C).astype(jnp.float32)
    b2 = conv2_b.reshape(1, Co).astype(jnp.float32)

    a = jnp.asarray(alpha, jnp.float32)
    wr0 = ((1.0 - a) * rgb0_w).astype(bf)
    wr1 = (a * rgb1_w).astype(bf)
    br = ((1.0 - a) * rgb0_b + a * rgb1_b).reshape(1, 3).astype(jnp.float32)

    out = pl.pallas_call(
        functools.partial(_fused_kernel, H=H, W=W),
        out_shape=jax.ShapeDtypeStruct((N, 4 * S, 3), jnp.float32),
        grid=(N,),
        in_specs=[
            pl.BlockSpec((None, S, C), lambda n: (n, 0, 0)),
            _const_spec(w1e), _const_spec(b1),
            _const_spec(w2), _const_spec(b2),
            _const_spec(wr0), _const_spec(wr1), _const_spec(br),
        ],
        out_specs=pl.BlockSpec((None, 4 * S, 3), lambda n: (n, 0, 0)),
        compiler_params=pltpu.CompilerParams(
            dimension_semantics=("parallel",)),
    )(xt, w1e, b1, w2, b2, wr0, wr1, br)

    o = out.reshape(N, 2, 2, H, W, 3)
    return o.transpose(0, 5, 3, 1, 4, 2).reshape(N, 3, 2 * H, 2 * W)
